# baseline (device time: 88813 ns/iter reference)
import functools

import jax
import jax.numpy as jnp
from jax import lax
from jax.experimental import pallas as pl
from jax.experimental.pallas import tpu as pltpu


def kernel(x, Wq, K_ext, V_ext, Wo):
    B, sq_loc, d_model = x.shape
    _, fq_loc = Wq.shape
    fo_loc, d_out = Wo.shape
    _, skv, hq, dh = K_ext.shape
    f_total = hq * dh
    n_dev = f_total // fq_loc
    n_hops = n_dev - 1

    def body(x_ref, wq_ref, k_ref, v_ref, wo_ref, out_ref,
             wqf, wof, wq_send, wq_recv, wo_send, wo_recv):
        my = lax.axis_index("i")
        left = lax.rem(my + n_dev - 1, n_dev)
        right = lax.rem(my + 1, n_dev)

        barrier = pltpu.get_barrier_semaphore()
        for nbr in (left, right):
            pl.semaphore_signal(barrier, inc=1, device_id=(nbr,),
                                device_id_type=pl.DeviceIdType.MESH)
        pl.semaphore_wait(barrier, 2)

        wqf[:, pl.ds(my * fq_loc, fq_loc)] = wq_ref[...].astype(jnp.bfloat16)
        wof[pl.ds(my * fo_loc, fo_loc), :] = wo_ref[...].astype(jnp.bfloat16)

        def hop_rdmas(h):
            o = lax.rem(my + n_dev - h, n_dev)
            rq = pltpu.make_async_remote_copy(
                src_ref=wqf.at[:, pl.ds(o * fq_loc, fq_loc)],
                dst_ref=wqf.at[:, pl.ds(o * fq_loc, fq_loc)],
                send_sem=wq_send.at[h], recv_sem=wq_recv.at[h],
                device_id=(right,), device_id_type=pl.DeviceIdType.MESH)
            ro = pltpu.make_async_remote_copy(
                src_ref=wof.at[pl.ds(o * fo_loc, fo_loc), :],
                dst_ref=wof.at[pl.ds(o * fo_loc, fo_loc), :],
                send_sem=wo_send.at[h], recv_sem=wo_recv.at[h],
                device_id=(right,), device_id_type=pl.DeviceIdType.MESH)
            return rq, ro

        hops = [hop_rdmas(h) for h in range(n_hops)]
        for h in range(n_hops):
            rq, ro = hops[h]
            rq.start()
            ro.start()
            rq.wait_recv()
            ro.wait_recv()
        for rq, ro in hops:
            rq.wait_send()
            ro.wait_send()

        wq_all = wqf[...]
        wo_all = wof[...]
        x_all = x_ref[...].astype(jnp.bfloat16)
        k_all = k_ref[...].astype(jnp.bfloat16)
        v_all = v_ref[...].astype(jnp.bfloat16)

        r = lax.broadcasted_iota(jnp.int32, (sq_loc, skv), 0)
        c = lax.broadcasted_iota(jnp.int32, (sq_loc, skv), 1)
        q_blk = my * (sq_loc // 64) + r // 64
        k_blk = c // 64
        mask = (q_blk == k_blk) | (k_blk == 0) | (lax.rem(q_blk + k_blk, 3) == 0)

        for b in range(B):
            q_mat = lax.dot_general(
                x_all[b], wq_all, (((1,), (0,)), ((), ())),
                preferred_element_type=jnp.float32,
            ).astype(jnp.bfloat16)
            ctx_parts = []
            for h in range(hq):
                q_h = q_mat[:, h * dh:(h + 1) * dh]
                k_h = k_all[b, :, h, :]
                v_h = v_all[b, :, h, :]
                s = lax.dot_general(
                    q_h, k_h, (((1,), (1,)), ((), ())),
                    preferred_element_type=jnp.float32,
                ) * 0.125
                s = jnp.where(mask, s, -1e9)
                s = s - jnp.max(s, axis=1, keepdims=True)
                w = jnp.exp(s)
                w = w / jnp.sum(w, axis=1, keepdims=True)
                ctx_parts.append(lax.dot_general(
                    w.astype(jnp.bfloat16), v_h, (((1,), (0,)), ((), ())),
                    preferred_element_type=jnp.float32,
                ).astype(jnp.bfloat16))
            ctx = jnp.concatenate(ctx_parts, axis=1)
            out_ref[b] = lax.dot_general(
                ctx, wo_all, (((1,), (0,)), ((), ())),
                preferred_element_type=jnp.float32,
            )

        @functools.partial(pl.run_scoped, exit_sem=pltpu.SemaphoreType.REGULAR)
        def _(exit_sem):
            for nbr in (left, right):
                pl.semaphore_signal(exit_sem, inc=1, device_id=(nbr,),
                                    device_id_type=pl.DeviceIdType.MESH)
            pl.semaphore_wait(exit_sem, 2)

    return pl.pallas_call(
        body,
        out_shape=jax.ShapeDtypeStruct((B, sq_loc, d_out), jnp.float32),
        in_specs=[pl.BlockSpec(memory_space=pltpu.VMEM)] * 5,
        out_specs=pl.BlockSpec(memory_space=pltpu.VMEM),
        scratch_shapes=[
            pltpu.VMEM((d_model, f_total), jnp.bfloat16),
            pltpu.VMEM((f_total, d_out), jnp.bfloat16),
            pltpu.SemaphoreType.DMA((n_hops,)),
            pltpu.SemaphoreType.DMA((n_hops,)),
            pltpu.SemaphoreType.DMA((n_hops,)),
            pltpu.SemaphoreType.DMA((n_hops,)),
        ],
        compiler_params=pltpu.CompilerParams(collective_id=0),
    )(x, Wq, K_ext, V_ext, Wo)


# device time: 34422 ns/iter; 2.5801x vs baseline; 2.5801x over previous
import functools

import jax
import jax.numpy as jnp
from jax import lax
from jax.experimental import pallas as pl
from jax.experimental.pallas import tpu as pltpu


def kernel(x, Wq, K_ext, V_ext, Wo):
    B, sq_loc, d_model = x.shape
    _, fq_loc = Wq.shape
    fo_loc, d_out = Wo.shape
    _, skv, hq, dh = K_ext.shape
    f_total = hq * dh
    n_dev = f_total // fq_loc
    hq_loc = fq_loc // dh
    n_steps = n_dev // 2

    def body(x_ref, wq_ref, k_ref, v_ref, wo_ref, out_ref,
             wqf, wof, k_stage, v_stage, k_hm_ref, v_hm_ref,
             rs_send, rs_recv, ls_send, ls_recv, kv_sems):
        my = lax.axis_index("i")
        left = lax.rem(my + n_dev - 1, n_dev)
        right = lax.rem(my + 1, n_dev)

        k_dma = pltpu.make_async_copy(k_ref, k_stage, kv_sems.at[0])
        v_dma = pltpu.make_async_copy(v_ref, v_stage, kv_sems.at[1])
        k_dma.start()
        v_dma.start()

        barrier = pltpu.get_barrier_semaphore()
        for nbr in (left, right):
            pl.semaphore_signal(barrier, inc=1, device_id=(nbr,),
                                device_id_type=pl.DeviceIdType.MESH)
        pl.semaphore_wait(barrier, 2)

        wqf[:, pl.ds(my * fq_loc, fq_loc)] = wq_ref[...].astype(jnp.bfloat16)
        wof[pl.ds(my * fo_loc, fo_loc), :] = wo_ref[...].astype(jnp.bfloat16)

        def blk_rdmas(o, dst, send_sems, recv_sems, h):
            rq = pltpu.make_async_remote_copy(
                src_ref=wqf.at[:, pl.ds(o * fq_loc, fq_loc)],
                dst_ref=wqf.at[:, pl.ds(o * fq_loc, fq_loc)],
                send_sem=send_sems.at[0, h], recv_sem=recv_sems.at[0, h],
                device_id=(dst,), device_id_type=pl.DeviceIdType.MESH)
            ro = pltpu.make_async_remote_copy(
                src_ref=wof.at[pl.ds(o * fo_loc, fo_loc), :],
                dst_ref=wof.at[pl.ds(o * fo_loc, fo_loc), :],
                send_sem=send_sems.at[1, h], recv_sem=recv_sems.at[1, h],
                device_id=(dst,), device_id_type=pl.DeviceIdType.MESH)
            return rq, ro

        r_hops = [blk_rdmas(lax.rem(my + n_dev - h, n_dev), right,
                            rs_send, rs_recv, h) for h in range(n_steps)]
        l_hops = [blk_rdmas(lax.rem(my + h, n_dev), left,
                            ls_send, ls_recv, h) for h in range(n_steps)]

        for r in (*r_hops[0], *l_hops[0]):
            r.start()

        x2 = x_ref[...].astype(jnp.bfloat16)
        k_dma.wait()
        v_dma.wait()
        for h in range(hq):
            k_hm_ref[:, h] = k_stage[:, :, h * dh:(h + 1) * dh].astype(jnp.bfloat16)
            v_hm_ref[:, h] = v_stage[:, :, h * dh:(h + 1) * dh].astype(jnp.bfloat16)

        r = lax.broadcasted_iota(jnp.int32, (sq_loc, skv), 0)
        c = lax.broadcasted_iota(jnp.int32, (sq_loc, skv), 1)
        q_blk = my * (sq_loc // 64) + r // 64
        k_blk = c // 64
        mask = (q_blk == k_blk) | (k_blk == 0) | (lax.rem(q_blk + k_blk, 3) == 0)

        def compute_block(o, wq_blk, wo_blk):
            qp = (lax.dot_general(
                x2, wq_blk, (((1,), (0,)), ((), ())),
                preferred_element_type=jnp.float32,
            ) * 0.125).astype(jnp.bfloat16)
            qh = jnp.transpose(
                qp.reshape(B, sq_loc, hq_loc, dh), (0, 2, 1, 3)
            ).reshape(B * hq_loc, sq_loc, dh)
            kb = k_hm_ref[:, pl.ds(o * hq_loc, hq_loc)].reshape(
                B * hq_loc, skv, dh)
            vb = v_hm_ref[:, pl.ds(o * hq_loc, hq_loc)].reshape(
                B * hq_loc, skv, dh)
            s = lax.dot_general(
                qh, kb, (((2,), (2,)), ((0,), (0,))),
                preferred_element_type=jnp.float32,
            )
            w = jnp.exp(jnp.where(mask[None], s, -1e9))
            w = (w / jnp.sum(w, axis=-1, keepdims=True)).astype(jnp.bfloat16)
            ctx = lax.dot_general(
                w, vb, (((2,), (1,)), ((0,), (0,))),
                preferred_element_type=jnp.float32,
            )
            ctx2 = jnp.transpose(
                ctx.reshape(B, hq_loc, sq_loc, dh), (0, 2, 1, 3)
            ).reshape(B * sq_loc, fq_loc).astype(jnp.bfloat16)
            return lax.dot_general(
                ctx2, wo_blk, (((1,), (0,)), ((), ())),
                preferred_element_type=jnp.float32,
            )

        acc = compute_block(my, wq_ref[...].astype(jnp.bfloat16),
                            wo_ref[...].astype(jnp.bfloat16))

        for step in range(n_steps):
            last = step == n_steps - 1
            br = lax.rem(my + n_dev - 1 - step, n_dev)
            bl = lax.rem(my + 1 + step, n_dev)
            r_hops[step][0].wait_recv()
            if step + 1 < n_steps:
                r_hops[step + 1][0].start()
            if not last:
                l_hops[step][1].wait_recv()
                if step + 1 < n_steps:
                    l_hops[step + 1][1].start()
                r_hops[step][1].wait_recv()
                if step + 1 < n_steps - 1:
                    r_hops[step + 1][1].start()
                l_hops[step][0].wait_recv()
                if step + 1 < n_steps - 1:
                    l_hops[step + 1][0].start()
            else:
                l_hops[step][1].wait_recv()
            acc += compute_block(br, wqf[:, pl.ds(br * fq_loc, fq_loc)],
                                 wof[pl.ds(br * fo_loc, fo_loc), :])
            if not last:
                acc += compute_block(bl, wqf[:, pl.ds(bl * fq_loc, fq_loc)],
                                     wof[pl.ds(bl * fo_loc, fo_loc), :])

        for h in range(n_steps):
            r_hops[h][0].wait_send()
            l_hops[h][1].wait_send()
            if h < n_steps - 1:
                r_hops[h][1].wait_send()
                l_hops[h][0].wait_send()

        out_ref[...] = acc

        @functools.partial(pl.run_scoped, exit_sem=pltpu.SemaphoreType.REGULAR)
        def _(exit_sem):
            for nbr in (left, right):
                pl.semaphore_signal(exit_sem, inc=1, device_id=(nbr,),
                                    device_id_type=pl.DeviceIdType.MESH)
            pl.semaphore_wait(exit_sem, 2)

    return pl.pallas_call(
        body,
        out_shape=jax.ShapeDtypeStruct((B * sq_loc, d_out), jnp.float32),
        in_specs=[
            pl.BlockSpec(memory_space=pltpu.VMEM),
            pl.BlockSpec(memory_space=pltpu.VMEM),
            pl.BlockSpec(memory_space=pl.ANY),
            pl.BlockSpec(memory_space=pl.ANY),
            pl.BlockSpec(memory_space=pltpu.VMEM),
        ],
        out_specs=pl.BlockSpec(memory_space=pltpu.VMEM),
        scratch_shapes=[
            pltpu.VMEM((d_model, f_total), jnp.bfloat16),
            pltpu.VMEM((f_total, d_out), jnp.bfloat16),
            pltpu.VMEM((B, skv, f_total), jnp.float32),
            pltpu.VMEM((B, skv, f_total), jnp.float32),
            pltpu.VMEM((B, hq, skv, dh), jnp.bfloat16),
            pltpu.VMEM((B, hq, skv, dh), jnp.bfloat16),
            pltpu.SemaphoreType.DMA((2, n_steps)),
            pltpu.SemaphoreType.DMA((2, n_steps)),
            pltpu.SemaphoreType.DMA((2, n_steps)),
            pltpu.SemaphoreType.DMA((2, n_steps)),
            pltpu.SemaphoreType.DMA((2,)),
        ],
        compiler_params=pltpu.CompilerParams(collective_id=0),
    )(x.reshape(B * sq_loc, d_model), Wq,
      K_ext.reshape(B, skv, f_total), V_ext.reshape(B, skv, f_total),
      Wo).reshape(B, sq_loc, d_out)
